# baseline (speedup 1.0000x reference)
import jax
import jax.numpy as jnp
from jax import lax
from jax.experimental import pallas as pl
from jax.experimental.pallas import tpu as pltpu
from jax.experimental.pallas import tpu_sc as plsc

B = 16384
D = 16
NC = 2
NS = 16
L = 16
NW = NC * NS
BPW = B // NW
G = 8
NCH = BPW // G
PAD = BPW + G


def _mf_body(ut_hbm, it_hbm, xt_hbm, out_hbm,
             uidx_v, iidx_v, u_st, i_st, out_v, sem0, sem1):
    wid = lax.axis_index("s") * NC + lax.axis_index("c")
    base = wid * BPW
    pltpu.sync_copy(xt_hbm.at[0, pl.ds(base, BPW)], uidx_v.at[pl.ds(0, BPW)])
    pltpu.sync_copy(xt_hbm.at[1, pl.ds(base, BPW)], iidx_v.at[pl.ds(0, BPW)])
    for ref in (uidx_v, iidx_v):
        keep = ref[pl.ds(BPW - L, L)]
        ref[pl.ds(BPW - G, L)] = jnp.zeros((L,), jnp.int32)
        ref[pl.ds(BPW - L, L)] = keep

    lanes = lax.iota(jnp.int32, L)
    slot = lanes & 7
    lo = lanes < G

    def fire(ch, parity):
        sem = [sem0, sem1][parity]
        rv = uidx_v[pl.ds(ch * G, L)]
        qv = iidx_v[pl.ds(ch * G, L)]
        for jj in range(G):
            r = pl.multiple_of((rv[jj] >> 7) << 7, 128)
            q = pl.multiple_of((qv[jj] >> 7) << 7, 128)
            pltpu.async_copy(
                ut_hbm.at[:, :, pl.ds(r, 128)], u_st.at[parity, jj], sem)
            pltpu.async_copy(
                it_hbm.at[:, :, pl.ds(q, 128)], i_st.at[parity, jj], sem)

    def drain_and_compute(ch, parity):
        sem = [sem0, sem1][parity]
        for jj in range(G):
            pltpu.make_async_copy(
                ut_hbm.at[:, :, pl.ds(0, 128)], u_st.at[parity, jj], sem
            ).wait()
            pltpu.make_async_copy(
                it_hbm.at[:, :, pl.ds(0, 128)], i_st.at[parity, jj], sem
            ).wait()

        rv = uidx_v[pl.ds(ch * G, L)]
        qv = iidx_v[pl.ds(ch * G, L)]
        ucols = rv & 127
        icols = qv & 127
        pvec = jnp.full((L,), parity, jnp.int32)
        acc = jnp.zeros((L,), jnp.float32)
        for b in range(2):
            for s in range(8):
                bb = jnp.full((L,), b, jnp.int32)
                ss = jnp.full((L,), s, jnp.int32)
                u = plsc.load_gather(u_st, [pvec, slot, bb, ss, ucols])
                v = plsc.load_gather(i_st, [pvec, slot, bb, ss, icols])
                acc = acc + u * v
        j0 = ch * G
        prev = out_v[pl.ds(j0, L)]
        out_v[pl.ds(j0, L)] = jnp.where(lo, acc, prev)

    fire(0, 0)

    def pair(p, carry):
        ch0 = p * 2

        @pl.when(ch0 + 1 < NCH)
        def _():
            fire(ch0 + 1, 1)

        drain_and_compute(ch0, 0)

        @pl.when(ch0 + 2 < NCH)
        def _():
            fire(ch0 + 2, 0)

        @pl.when(ch0 + 1 < NCH)
        def _():
            drain_and_compute(ch0 + 1, 1)

        return carry

    lax.fori_loop(0, NCH // 2, pair, 0)
    pltpu.sync_copy(out_v.at[pl.ds(0, BPW)], out_hbm.at[pl.ds(base, BPW)])


@jax.jit
def kernel(x, uid_table, iid_table):
    ut = uid_table.T.reshape(2, 8, uid_table.shape[0])
    it = iid_table.T.reshape(2, 8, iid_table.shape[0])
    k = pl.kernel(
        _mf_body,
        out_type=jax.ShapeDtypeStruct((B,), jnp.float32),
        mesh=plsc.VectorSubcoreMesh(core_axis_name="c", subcore_axis_name="s"),
        scratch_types=[
            pltpu.VMEM((PAD,), jnp.int32),
            pltpu.VMEM((PAD,), jnp.int32),
            pltpu.VMEM((2, G, 2, 8, 128), jnp.float32),
            pltpu.VMEM((2, G, 2, 8, 128), jnp.float32),
            pltpu.VMEM((PAD,), jnp.float32),
            pltpu.SemaphoreType.DMA,
            pltpu.SemaphoreType.DMA,
        ],
        compiler_params=pltpu.CompilerParams(needs_layout_passes=False),
    )
    return k(ut, it, x.T)


# speedup vs baseline: 1.0573x; 1.0573x over previous
import jax
import jax.numpy as jnp
from jax import lax
from jax.experimental import pallas as pl
from jax.experimental.pallas import tpu as pltpu
from jax.experimental.pallas import tpu_sc as plsc

B = 16384
D = 16
NC = 2
NS = 16
L = 16
NW = NC * NS
BPW = B // NW
G = 8
NCH = BPW // G
PAD = BPW + G


def _mf_body(ut_hbm, it_hbm, xt_hbm, out_hbm,
             uidx_v, iidx_v, u_st, i_st, out_v, sem0, sem1, sem2):
    wid = lax.axis_index("s") * NC + lax.axis_index("c")
    base = wid * BPW
    pltpu.sync_copy(xt_hbm.at[0, pl.ds(base, BPW)], uidx_v.at[pl.ds(0, BPW)])
    pltpu.sync_copy(xt_hbm.at[1, pl.ds(base, BPW)], iidx_v.at[pl.ds(0, BPW)])
    for ref in (uidx_v, iidx_v):
        keep = ref[pl.ds(BPW - L, L)]
        ref[pl.ds(BPW - G, L)] = jnp.zeros((L,), jnp.int32)
        ref[pl.ds(BPW - L, L)] = keep

    lanes = lax.iota(jnp.int32, L)
    slot = lanes & 7
    lo = lanes < G

    def fire(ch, parity):
        sem = [sem0, sem1, sem2][parity]
        rv = uidx_v[pl.ds(ch * G, L)]
        qv = iidx_v[pl.ds(ch * G, L)]
        for jj in range(G):
            r = pl.multiple_of((rv[jj] >> 7) << 7, 128)
            q = pl.multiple_of((qv[jj] >> 7) << 7, 128)
            pltpu.async_copy(
                ut_hbm.at[:, :, pl.ds(r, 128)], u_st.at[parity, jj], sem)
            pltpu.async_copy(
                it_hbm.at[:, :, pl.ds(q, 128)], i_st.at[parity, jj], sem)

    def drain_and_compute(ch, parity):
        sem = [sem0, sem1, sem2][parity]
        for jj in range(G):
            pltpu.make_async_copy(
                ut_hbm.at[:, :, pl.ds(0, 128)], u_st.at[parity, jj], sem
            ).wait()
            pltpu.make_async_copy(
                it_hbm.at[:, :, pl.ds(0, 128)], i_st.at[parity, jj], sem
            ).wait()

        rv = uidx_v[pl.ds(ch * G, L)]
        qv = iidx_v[pl.ds(ch * G, L)]
        ucols = rv & 127
        icols = qv & 127
        pvec = jnp.full((L,), parity, jnp.int32)
        acc = jnp.zeros((L,), jnp.float32)
        for b in range(2):
            for s in range(8):
                bb = jnp.full((L,), b, jnp.int32)
                ss = jnp.full((L,), s, jnp.int32)
                u = plsc.load_gather(u_st, [pvec, slot, bb, ss, ucols])
                v = plsc.load_gather(i_st, [pvec, slot, bb, ss, icols])
                acc = acc + u * v
        j0 = ch * G
        prev = out_v[pl.ds(j0, L)]
        out_v[pl.ds(j0, L)] = jnp.where(lo, acc, prev)

    fire(0, 0)
    fire(1, 1)

    def triple(p, carry):
        ch0 = p * 3
        for k in range(3):
            ch = ch0 + k

            @pl.when(ch + 2 < NCH)
            def _():
                fire(ch + 2, (k + 2) % 3)

            @pl.when(ch < NCH)
            def _():
                drain_and_compute(ch, k)

        return carry

    lax.fori_loop(0, (NCH + 2) // 3, triple, 0)
    pltpu.sync_copy(out_v.at[pl.ds(0, BPW)], out_hbm.at[pl.ds(base, BPW)])


@jax.jit
def kernel(x, uid_table, iid_table):
    ut = uid_table.T.reshape(2, 8, uid_table.shape[0])
    it = iid_table.T.reshape(2, 8, iid_table.shape[0])
    k = pl.kernel(
        _mf_body,
        out_type=jax.ShapeDtypeStruct((B,), jnp.float32),
        mesh=plsc.VectorSubcoreMesh(core_axis_name="c", subcore_axis_name="s"),
        scratch_types=[
            pltpu.VMEM((PAD,), jnp.int32),
            pltpu.VMEM((PAD,), jnp.int32),
            pltpu.VMEM((3, G, 2, 8, 128), jnp.float32),
            pltpu.VMEM((3, G, 2, 8, 128), jnp.float32),
            pltpu.VMEM((PAD,), jnp.float32),
            pltpu.SemaphoreType.DMA,
            pltpu.SemaphoreType.DMA,
            pltpu.SemaphoreType.DMA,
        ],
        compiler_params=pltpu.CompilerParams(needs_layout_passes=False),
    )
    return k(ut, it, x.T)
